# baseline (speedup 1.0000x reference)
import functools

import jax
import jax.numpy as jnp
from jax import lax
from jax.experimental import pallas as pl
from jax.experimental.pallas import tpu as pltpu
from jax.experimental.pallas import tpu_sc as plsc

N = 10000
E = 320000
D = 128

NC = 2
NS = 16
NW = NC * NS
CH = 128

NP = 10240
ROWS_PER_TILE = NP // NS
K_PER_W = 80
NCHUNKS = K_PER_W * NW
E_PAD = NCHUNKS * CH
NBUF = 4


def _mesh():
    return plsc.VectorSubcoreMesh(core_axis_name="c", subcore_axis_name="s")


def _deg_call(dstc, zeros1d, ones1d):
    @functools.partial(
        pl.kernel,
        out_type=jax.ShapeDtypeStruct((NC, NP), jnp.float32),
        mesh=_mesh(),
        scratch_types=[
            pltpu.VMEM((K_PER_W, CH), jnp.int32),
            pltpu.VMEM((CH,), jnp.float32),
            pltpu.VMEM_SHARED((NP,), jnp.float32),
        ],
    )
    def k(dst_hbm, z_hbm, ones_hbm, out_hbm, dst_v, ones_v, acc):
        c = lax.axis_index("c")
        s = lax.axis_index("s")
        wid = c * NS + s
        pltpu.sync_copy(z_hbm.at[pl.ds(s * ROWS_PER_TILE, ROWS_PER_TILE)],
                        acc.at[pl.ds(s * ROWS_PER_TILE, ROWS_PER_TILE)])
        pltpu.sync_copy(dst_hbm.at[pl.ds(wid * K_PER_W, K_PER_W)], dst_v)
        pltpu.sync_copy(ones_hbm, ones_v)
        plsc.subcore_barrier()

        def body(j, carry):
            pltpu.sync_copy(ones_v, acc.at[dst_v.at[j]], add=True)
            return carry

        lax.fori_loop(0, K_PER_W, body, 0)
        plsc.subcore_barrier()
        pltpu.sync_copy(acc.at[pl.ds(s * ROWS_PER_TILE, ROWS_PER_TILE)],
                        out_hbm.at[c, pl.ds(s * ROWS_PER_TILE, ROWS_PER_TILE)])

    return k(dstc, zeros1d, ones1d)


def _prop_impl(y, srcc, dstc, zeros2d, hh, dt=jnp.float32):
    KT = K_PER_W
    LAG = NBUF // 2

    @functools.partial(
        pl.kernel,
        out_type=jax.ShapeDtypeStruct((NC, NP, hh), dt),
        mesh=_mesh(),
        scratch_types=(
            [pltpu.VMEM_SHARED((NP, hh), dt),
             pltpu.VMEM((KT, CH), jnp.int32),
             pltpu.VMEM((KT, CH), jnp.int32)]
            + [pltpu.VMEM((CH, hh), dt) for _ in range(NBUF)]
            + [pltpu.SemaphoreType.DMA for _ in range(NBUF)]
            + [pltpu.SemaphoreType.DMA for _ in range(NBUF)]
        ),
        compiler_params=pltpu.CompilerParams(use_tc_tiling_on_sc=False),
    )
    def k(y_hbm, src_hbm, dst_hbm, z_hbm, out_hbm, acc, src_v, dst_v, *rest):
        rows = rest[:NBUF]
        semg = rest[NBUF:2 * NBUF]
        sems = rest[2 * NBUF:]
        c = lax.axis_index("c")
        s = lax.axis_index("s")
        r0 = s * ROWS_PER_TILE
        ysrc = y_hbm
        base = (c * NS + s) * KT
        pltpu.sync_copy(z_hbm.at[pl.ds(r0, ROWS_PER_TILE)],
                        acc.at[pl.ds(r0, ROWS_PER_TILE)])
        pltpu.sync_copy(src_hbm.at[pl.ds(base, KT)], src_v)
        pltpu.sync_copy(dst_hbm.at[pl.ds(base, KT)], dst_v)
        plsc.subcore_barrier()

        def gather(j, b):
            pltpu.async_copy(ysrc.at[src_v.at[j]], rows[b], semg[b])

        def wait_gather(j, b):
            pltpu.make_async_copy(ysrc.at[src_v.at[j]], rows[b], semg[b]).wait()

        def scatter(j, b):
            pltpu.async_copy(rows[b], acc.at[dst_v.at[j]], sems[b], add=True)

        def wait_scatter(j, b):
            pltpu.make_async_copy(rows[b], acc.at[dst_v.at[j]], sems[b]).wait()

        for b in range(NBUF):
            gather(b, b)

        def body(jo, carry):
            for b in range(NBUF):
                j = jo * NBUF + b
                wait_gather(j, b)
                scatter(j, b)
                n = j + LAG
                bn = (b + LAG) % NBUF

                @pl.when(jnp.logical_and(n >= NBUF, n < KT))
                def _():
                    wait_scatter(n - NBUF, bn)
                    gather(n, bn)
            return carry

        lax.fori_loop(0, KT // NBUF, body, 0)
        for b in range(NBUF):
            wait_scatter(KT - NBUF + b, (KT - NBUF + b) % NBUF)
        plsc.subcore_barrier()
        pltpu.sync_copy(acc.at[pl.ds(r0, ROWS_PER_TILE)],
                        out_hbm.at[c, pl.ds(r0, ROWS_PER_TILE)])

    return k(y, srcc, dstc, zeros2d)


_TC_GRID_BN = 2048


def _tc_mm(x, w1):
    def body(x_ref, w_ref, o_ref):
        o_ref[...] = jnp.dot(x_ref[...], w_ref[...],
                             preferred_element_type=jnp.float32)

    bn = _TC_GRID_BN
    return pl.pallas_call(
        body,
        grid=(NP // bn,),
        in_specs=[
            pl.BlockSpec((bn, D), lambda i: (i, 0)),
            pl.BlockSpec((D, D), lambda i: (0, 0)),
        ],
        out_specs=pl.BlockSpec((bn, D), lambda i: (i, 0)),
        out_shape=jax.ShapeDtypeStruct((NP, D), jnp.float32),
    )(x, w1)


def _tc_scale(xw, d0, d1):
    def body(xw_ref, d0_ref, d1_ref, dinv_ref, y_ref):
        dinv = lax.rsqrt(d0_ref[...] + d1_ref[...] + 1.0)
        dinv_ref[...] = dinv
        y_ref[...] = (dinv * xw_ref[...]).astype(jnp.bfloat16)

    bn = _TC_GRID_BN
    return pl.pallas_call(
        body,
        grid=(NP // bn,),
        in_specs=[
            pl.BlockSpec((bn, D), lambda i: (i, 0)),
            pl.BlockSpec((bn, 1), lambda i: (i, 0)),
            pl.BlockSpec((bn, 1), lambda i: (i, 0)),
        ],
        out_specs=[
            pl.BlockSpec((bn, 1), lambda i: (i, 0)),
            pl.BlockSpec((bn, D), lambda i: (i, 0)),
        ],
        out_shape=[
            jax.ShapeDtypeStruct((NP, 1), jnp.float32),
            jax.ShapeDtypeStruct((NP, D), jnp.bfloat16),
        ],
    )(xw, d0, d1)


def _tc_mid(p0, p1, y, dinv, b, w, h, hout):
    def body(p0_ref, p1_ref, y_ref, dinv_ref, b_ref, w_ref, yout_ref):
        t = (p0_ref[...].astype(jnp.float32) + p1_ref[...].astype(jnp.float32)
             + y_ref[...].astype(jnp.float32))
        s = dinv_ref[...] * t + b_ref[...]
        hmid = jnp.where(s >= 0, s, 0.2 * s)
        yout_ref[...] = dinv_ref[...] * jnp.dot(hmid, w_ref[...],
                                                preferred_element_type=jnp.float32)

    bn = _TC_GRID_BN
    return pl.pallas_call(
        body,
        grid=(NP // bn,),
        in_specs=[
            pl.BlockSpec((bn, h), lambda i: (i, 0)),
            pl.BlockSpec((bn, h), lambda i: (i, 0)),
            pl.BlockSpec((bn, h), lambda i: (i, 0)),
            pl.BlockSpec((bn, 1), lambda i: (i, 0)),
            pl.BlockSpec((1, h), lambda i: (0, 0)),
            pl.BlockSpec((h, hout), lambda i: (0, 0)),
        ],
        out_specs=pl.BlockSpec((bn, hout), lambda i: (i, 0)),
        out_shape=jax.ShapeDtypeStruct((NP, hout), jnp.float32),
    )(p0, p1, y, dinv, b, w)


def _tc_final(p0, p1, y, dinv, b, h):
    def body(p0_ref, p1_ref, y_ref, dinv_ref, b_ref, out_ref):
        out_ref[...] = (dinv_ref[...] * (p0_ref[...] + p1_ref[...] + y_ref[...])
                        + b_ref[...])

    bn = _TC_GRID_BN
    return pl.pallas_call(
        body,
        grid=(NP // bn,),
        in_specs=[
            pl.BlockSpec((bn, h), lambda i: (i, 0)),
            pl.BlockSpec((bn, h), lambda i: (i, 0)),
            pl.BlockSpec((bn, h), lambda i: (i, 0)),
            pl.BlockSpec((bn, 1), lambda i: (i, 0)),
            pl.BlockSpec((1, h), lambda i: (0, 0)),
        ],
        out_specs=pl.BlockSpec((bn, h), lambda i: (i, 0)),
        out_shape=jax.ShapeDtypeStruct((NP, h), jnp.float32),
    )(p0, p1, y, dinv, b)


def kernel(x, edge_index, W1, b1, W2, b2, W3, b3):
    H1 = W1.shape[1]
    H2 = W2.shape[1]
    C = W3.shape[1]

    src = edge_index[0]
    dst = edge_index[1]
    pad_e = E_PAD - E
    pad_idx = jnp.full((pad_e,), N, dtype=jnp.int32)
    srcc = jnp.concatenate([src, pad_idx]).reshape(NCHUNKS, CH)
    dstc = jnp.concatenate([dst, pad_idx]).reshape(NCHUNKS, CH)

    xp = jnp.zeros((NP, D), jnp.float32).at[:N].set(x)
    ones1d = jnp.ones((CH,), jnp.float32)
    zeros1d = jnp.zeros((NP,), jnp.float32)
    zeros128 = jnp.zeros((NP, D), jnp.bfloat16)
    zerosH2 = jnp.zeros((NP, H2), jnp.float32)
    zerosC = jnp.zeros((NP, C), jnp.float32)

    deg = _deg_call(dstc, zeros1d, ones1d)
    xw1 = _tc_mm(xp, W1)
    d0 = deg[0].reshape(NP, 1)
    d1 = deg[1].reshape(NP, 1)
    dinv, y1 = _tc_scale(xw1, d0, d1)

    p = _prop_impl(y1, srcc, dstc, zeros128, H1, jnp.bfloat16)
    y2 = _tc_mid(p[0], p[1], y1, dinv, b1.reshape(1, H1), W2, H1, H2)

    p = _prop_impl(y2, srcc, dstc, zerosH2, H2)
    y3 = _tc_mid(p[0], p[1], y2, dinv, b2.reshape(1, H2), W3, H2, C)

    p = _prop_impl(y3, srcc, dstc, zerosC, C)
    out = _tc_final(p[0], p[1], y3, dinv, b3.reshape(1, C), C)
    return out[:N]


# speedup vs baseline: 1.5786x; 1.5786x over previous
import functools

import jax
import jax.numpy as jnp
from jax import lax
from jax.experimental import pallas as pl
from jax.experimental.pallas import tpu as pltpu
from jax.experimental.pallas import tpu_sc as plsc

N = 10000
E = 320000
D = 128

NC = 2
NS = 16
NW = NC * NS
CH = 128

NP = 10240
ROWS_PER_TILE = NP // NS
K_PER_W = 80
NCHUNKS = K_PER_W * NW
E_PAD = NCHUNKS * CH
NBUF = 4


def _mesh():
    return plsc.VectorSubcoreMesh(core_axis_name="c", subcore_axis_name="s")


def _deg_call(dstc, zeros1d, ones1d):
    @functools.partial(
        pl.kernel,
        out_type=jax.ShapeDtypeStruct((NC, NP), jnp.float32),
        mesh=_mesh(),
        scratch_types=[
            pltpu.VMEM((K_PER_W, CH), jnp.int32),
            pltpu.VMEM((CH,), jnp.float32),
            pltpu.VMEM_SHARED((NP,), jnp.float32),
        ],
    )
    def k(dst_hbm, z_hbm, ones_hbm, out_hbm, dst_v, ones_v, acc):
        c = lax.axis_index("c")
        s = lax.axis_index("s")
        wid = c * NS + s
        pltpu.sync_copy(z_hbm.at[pl.ds(s * ROWS_PER_TILE, ROWS_PER_TILE)],
                        acc.at[pl.ds(s * ROWS_PER_TILE, ROWS_PER_TILE)])
        pltpu.sync_copy(dst_hbm.at[pl.ds(wid * K_PER_W, K_PER_W)], dst_v)
        pltpu.sync_copy(ones_hbm, ones_v)
        plsc.subcore_barrier()

        def body(j, carry):
            pltpu.sync_copy(ones_v, acc.at[dst_v.at[j]], add=True)
            return carry

        lax.fori_loop(0, K_PER_W, body, 0)
        plsc.subcore_barrier()
        pltpu.sync_copy(acc.at[pl.ds(s * ROWS_PER_TILE, ROWS_PER_TILE)],
                        out_hbm.at[c, pl.ds(s * ROWS_PER_TILE, ROWS_PER_TILE)])

    return k(dstc, zeros1d, ones1d)


def _prop_impl(y, srcc, dstc, zeros2d, hh, dt=jnp.float32):
    KT = K_PER_W
    LAG = NBUF // 2

    @functools.partial(
        pl.kernel,
        out_type=jax.ShapeDtypeStruct((NC, NP, hh), dt),
        mesh=_mesh(),
        scratch_types=(
            [pltpu.VMEM_SHARED((NP, hh), dt),
             pltpu.VMEM((KT, CH), jnp.int32),
             pltpu.VMEM((KT, CH), jnp.int32)]
            + [pltpu.VMEM((CH, hh), dt) for _ in range(NBUF)]
            + [pltpu.SemaphoreType.DMA for _ in range(NBUF)]
            + [pltpu.SemaphoreType.DMA for _ in range(NBUF)]
        ),
        compiler_params=pltpu.CompilerParams(use_tc_tiling_on_sc=False),
    )
    def k(y_hbm, src_hbm, dst_hbm, z_hbm, out_hbm, acc, src_v, dst_v, *rest):
        rows = rest[:NBUF]
        semg = rest[NBUF:2 * NBUF]
        sems = rest[2 * NBUF:]
        c = lax.axis_index("c")
        s = lax.axis_index("s")
        r0 = s * ROWS_PER_TILE
        ysrc = y_hbm
        base = (c * NS + s) * KT
        pltpu.sync_copy(z_hbm.at[pl.ds(r0, ROWS_PER_TILE)],
                        acc.at[pl.ds(r0, ROWS_PER_TILE)])
        pltpu.sync_copy(src_hbm.at[pl.ds(base, KT)], src_v)
        pltpu.sync_copy(dst_hbm.at[pl.ds(base, KT)], dst_v)
        plsc.subcore_barrier()

        def gather(j, b):
            pltpu.async_copy(ysrc.at[src_v.at[j]], rows[b], semg[b])

        def wait_gather(j, b):
            pltpu.make_async_copy(ysrc.at[src_v.at[j]], rows[b], semg[b]).wait()

        def scatter(j, b):
            pltpu.async_copy(rows[b], acc.at[dst_v.at[j]], sems[b], add=True)

        def wait_scatter(j, b):
            pltpu.make_async_copy(rows[b], acc.at[dst_v.at[j]], sems[b]).wait()

        for b in range(NBUF):
            gather(b, b)

        def body(jo, carry):
            for b in range(NBUF):
                j = jo * NBUF + b
                wait_gather(j, b)
                scatter(j, b)
                n = j + LAG
                bn = (b + LAG) % NBUF

                @pl.when(jnp.logical_and(n >= NBUF, n < KT))
                def _():
                    wait_scatter(n - NBUF, bn)
                    gather(n, bn)
            return carry

        lax.fori_loop(0, KT // NBUF, body, 0)
        for b in range(NBUF):
            wait_scatter(KT - NBUF + b, (KT - NBUF + b) % NBUF)
        plsc.subcore_barrier()
        pltpu.sync_copy(acc.at[pl.ds(r0, ROWS_PER_TILE)],
                        out_hbm.at[c, pl.ds(r0, ROWS_PER_TILE)])

    return k(y, srcc, dstc, zeros2d)


_TC_GRID_BN = 2048


def _tc_mm(x, w1):
    def body(x_ref, w_ref, o_ref):
        o_ref[...] = jnp.dot(x_ref[...], w_ref[...],
                             preferred_element_type=jnp.float32)

    bn = _TC_GRID_BN
    return pl.pallas_call(
        body,
        grid=(NP // bn,),
        in_specs=[
            pl.BlockSpec((bn, D), lambda i: (i, 0)),
            pl.BlockSpec((D, D), lambda i: (0, 0)),
        ],
        out_specs=pl.BlockSpec((bn, D), lambda i: (i, 0)),
        out_shape=jax.ShapeDtypeStruct((NP, D), jnp.float32),
    )(x, w1)


def _tc_scale(xw, d0, d1):
    def body(xw_ref, d0_ref, d1_ref, dinv_ref, y_ref):
        dinv = lax.rsqrt(d0_ref[...] + d1_ref[...] + 1.0)
        dinv_ref[...] = dinv
        y_ref[...] = (dinv * xw_ref[...]).astype(jnp.bfloat16)

    bn = _TC_GRID_BN
    return pl.pallas_call(
        body,
        grid=(NP // bn,),
        in_specs=[
            pl.BlockSpec((bn, D), lambda i: (i, 0)),
            pl.BlockSpec((bn, 1), lambda i: (i, 0)),
            pl.BlockSpec((bn, 1), lambda i: (i, 0)),
        ],
        out_specs=[
            pl.BlockSpec((bn, 1), lambda i: (i, 0)),
            pl.BlockSpec((bn, D), lambda i: (i, 0)),
        ],
        out_shape=[
            jax.ShapeDtypeStruct((NP, 1), jnp.float32),
            jax.ShapeDtypeStruct((NP, D), jnp.bfloat16),
        ],
    )(xw, d0, d1)


def _tc_mid(p0, p1, y, dinv, b, w, h, hout):
    def body(p0_ref, p1_ref, y_ref, dinv_ref, b_ref, w_ref, yout_ref):
        t = (p0_ref[...].astype(jnp.float32) + p1_ref[...].astype(jnp.float32)
             + y_ref[...].astype(jnp.float32))
        s = dinv_ref[...] * t + b_ref[...]
        hmid = jnp.where(s >= 0, s, 0.2 * s)
        yout_ref[...] = dinv_ref[...] * jnp.dot(hmid, w_ref[...],
                                                preferred_element_type=jnp.float32)

    bn = _TC_GRID_BN
    return pl.pallas_call(
        body,
        grid=(NP // bn,),
        in_specs=[
            pl.BlockSpec((bn, h), lambda i: (i, 0)),
            pl.BlockSpec((bn, h), lambda i: (i, 0)),
            pl.BlockSpec((bn, h), lambda i: (i, 0)),
            pl.BlockSpec((bn, 1), lambda i: (i, 0)),
            pl.BlockSpec((1, h), lambda i: (0, 0)),
            pl.BlockSpec((h, hout), lambda i: (0, 0)),
        ],
        out_specs=pl.BlockSpec((bn, hout), lambda i: (i, 0)),
        out_shape=jax.ShapeDtypeStruct((NP, hout), jnp.float32),
    )(p0, p1, y, dinv, b, w)


def _tc_final(p0, p1, y, dinv, b, h):
    def body(p0_ref, p1_ref, y_ref, dinv_ref, b_ref, out_ref):
        out_ref[...] = (dinv_ref[...] * (p0_ref[...] + p1_ref[...] + y_ref[...])
                        + b_ref[...])

    bn = _TC_GRID_BN
    return pl.pallas_call(
        body,
        grid=(NP // bn,),
        in_specs=[
            pl.BlockSpec((bn, h), lambda i: (i, 0)),
            pl.BlockSpec((bn, h), lambda i: (i, 0)),
            pl.BlockSpec((bn, h), lambda i: (i, 0)),
            pl.BlockSpec((bn, 1), lambda i: (i, 0)),
            pl.BlockSpec((1, h), lambda i: (0, 0)),
        ],
        out_specs=pl.BlockSpec((bn, h), lambda i: (i, 0)),
        out_shape=jax.ShapeDtypeStruct((NP, h), jnp.float32),
    )(p0, p1, y, dinv, b)


def kernel(x, edge_index, W1, b1, W2, b2, W3, b3):
    H1 = W1.shape[1]
    H2 = W2.shape[1]
    C = W3.shape[1]

    src = edge_index[0]
    dst = edge_index[1]
    pad_e = E_PAD - E
    pad_idx = N + (jnp.arange(pad_e, dtype=jnp.int32) % (NP - N))
    srcc = jnp.concatenate([src, pad_idx]).reshape(NCHUNKS, CH)
    dstc = jnp.concatenate([dst, pad_idx]).reshape(NCHUNKS, CH)

    xp = jnp.zeros((NP, D), jnp.float32).at[:N].set(x)
    ones1d = jnp.ones((CH,), jnp.float32)
    zeros1d = jnp.zeros((NP,), jnp.float32)
    zeros128 = jnp.zeros((NP, D), jnp.bfloat16)
    zerosH2 = jnp.zeros((NP, H2), jnp.float32)
    zerosC = jnp.zeros((NP, C), jnp.float32)

    deg = _deg_call(dstc, zeros1d, ones1d)
    xw1 = _tc_mm(xp, W1)
    d0 = deg[0].reshape(NP, 1)
    d1 = deg[1].reshape(NP, 1)
    dinv, y1 = _tc_scale(xw1, d0, d1)

    p = _prop_impl(y1, srcc, dstc, zeros128, H1, jnp.bfloat16)
    y2 = _tc_mid(p[0], p[1], y1, dinv, b1.reshape(1, H1), W2, H1, H2)

    p = _prop_impl(y2, srcc, dstc, zerosH2, H2)
    y3 = _tc_mid(p[0], p[1], y2, dinv, b2.reshape(1, H2), W3, H2, C)

    p = _prop_impl(y3, srcc, dstc, zerosC, C)
    out = _tc_final(p[0], p[1], y3, dinv, b3.reshape(1, C), C)
    return out[:N]


# speedup vs baseline: 1.6781x; 1.0630x over previous
import functools

import jax
import jax.numpy as jnp
from jax import lax
from jax.experimental import pallas as pl
from jax.experimental.pallas import tpu as pltpu
from jax.experimental.pallas import tpu_sc as plsc

N = 10000
E = 320000
D = 128

NC = 2
NS = 16
NW = NC * NS
CH = 128

NP = 10240
ROWS_PER_TILE = NP // NS
NROWS = E // CH
KT_FULL = 80
KT_LAST = NROWS - 31 * KT_FULL
NITER = KT_FULL
NBUF = 4
LAG = NBUF // 2


def _mesh():
    return plsc.VectorSubcoreMesh(core_axis_name="c", subcore_axis_name="s")


def _sc_params():
    return pltpu.CompilerParams(use_tc_tiling_on_sc=False)


def _tile_work(c, s):
    w = c * NS + s
    base = w * KT_FULL
    kt = jnp.where(w < NW - 1, KT_FULL, KT_LAST)
    return w, base, kt


def _load_idx(e2d, base, w, dst_v):
    @pl.when(w < NW - 1)
    def _():
        pltpu.sync_copy(e2d.at[pl.ds(base, KT_FULL)], dst_v)

    @pl.when(w == NW - 1)
    def _():
        pltpu.sync_copy(e2d.at[pl.ds(base, KT_LAST)],
                        dst_v.at[pl.ds(0, KT_LAST)])


def _deg_call(e3, zdeg, ones2):
    @functools.partial(
        pl.kernel,
        out_type=jax.ShapeDtypeStruct((NC, NP, 1), jnp.float32),
        mesh=_mesh(),
        scratch_types=[
            pltpu.VMEM((KT_FULL, CH), jnp.int32),
            pltpu.VMEM((CH, 1), jnp.float32),
            pltpu.VMEM_SHARED((NP, 1), jnp.float32),
        ],
        compiler_params=_sc_params(),
    )
    def k(e_hbm, z_hbm, ones_hbm, out_hbm, dst_v, ones_v, acc):
        c = lax.axis_index("c")
        s = lax.axis_index("s")
        w, base, kt = _tile_work(c, s)
        r0 = s * ROWS_PER_TILE
        pltpu.sync_copy(z_hbm.at[pl.ds(r0, ROWS_PER_TILE)],
                        acc.at[pl.ds(r0, ROWS_PER_TILE)])
        _load_idx(e_hbm.at[1], base, w, dst_v)
        pltpu.sync_copy(ones_hbm, ones_v)
        plsc.subcore_barrier()

        def body(j, carry):
            pltpu.sync_copy(ones_v, acc.at[dst_v.at[j]], add=True)
            return carry

        lax.fori_loop(0, kt, body, 0)
        plsc.subcore_barrier()
        pltpu.sync_copy(acc.at[pl.ds(r0, ROWS_PER_TILE)],
                        out_hbm.at[c, pl.ds(r0, ROWS_PER_TILE)])

    return k(e3, zdeg, ones2)


def _prop_impl(y, e3, zeros2d, hh, dt=jnp.float32):
    @functools.partial(
        pl.kernel,
        out_type=jax.ShapeDtypeStruct((NC, NP, hh), dt),
        mesh=_mesh(),
        scratch_types=(
            [pltpu.VMEM_SHARED((NP, hh), dt),
             pltpu.VMEM((KT_FULL, CH), jnp.int32),
             pltpu.VMEM((KT_FULL, CH), jnp.int32)]
            + [pltpu.VMEM((CH, hh), dt) for _ in range(NBUF)]
            + [pltpu.SemaphoreType.DMA for _ in range(NBUF)]
            + [pltpu.SemaphoreType.DMA for _ in range(NBUF)]
        ),
        compiler_params=_sc_params(),
    )
    def k(y_hbm, e_hbm, z_hbm, out_hbm, acc, src_v, dst_v, *rest):
        rows = rest[:NBUF]
        semg = rest[NBUF:2 * NBUF]
        sems = rest[2 * NBUF:]
        c = lax.axis_index("c")
        s = lax.axis_index("s")
        w, base, kt = _tile_work(c, s)
        r0 = s * ROWS_PER_TILE
        pltpu.sync_copy(z_hbm.at[pl.ds(r0, ROWS_PER_TILE)],
                        acc.at[pl.ds(r0, ROWS_PER_TILE)])
        _load_idx(e_hbm.at[0], base, w, src_v)
        _load_idx(e_hbm.at[1], base, w, dst_v)
        plsc.subcore_barrier()

        def gather(j, b):
            pltpu.async_copy(y_hbm.at[src_v.at[j]], rows[b], semg[b])

        def wait_gather(j, b):
            pltpu.make_async_copy(y_hbm.at[src_v.at[j]], rows[b], semg[b]).wait()

        def scatter(j, b):
            pltpu.async_copy(rows[b], acc.at[dst_v.at[j]], sems[b], add=True)

        def wait_scatter(j, b):
            pltpu.make_async_copy(rows[b], acc.at[dst_v.at[j]], sems[b]).wait()

        for b in range(NBUF):
            gather(b, b)

        def body(jo, carry):
            for b in range(NBUF):
                j = jo * NBUF + b

                @pl.when(j < kt)
                def _():
                    wait_gather(j, b)
                    scatter(j, b)

                n = j + LAG
                bn = (b + LAG) % NBUF

                @pl.when(jnp.logical_and(n >= NBUF, n < kt))
                def _():
                    wait_scatter(n - NBUF, bn)
                    gather(n, bn)
            return carry

        lax.fori_loop(0, NITER // NBUF, body, 0)
        for b in range(NBUF):
            wait_scatter(0, b)
        plsc.subcore_barrier()
        pltpu.sync_copy(acc.at[pl.ds(r0, ROWS_PER_TILE)],
                        out_hbm.at[c, pl.ds(r0, ROWS_PER_TILE)])

    return k(y, e3, zeros2d)


_TC_GRID_BN = 2048


def _tc_mm(x, w1):
    def body(x_ref, w_ref, o_ref):
        o_ref[...] = jnp.dot(x_ref[...], w_ref[...],
                             preferred_element_type=jnp.float32)

    bn = _TC_GRID_BN
    return pl.pallas_call(
        body,
        grid=(NP // bn,),
        in_specs=[
            pl.BlockSpec((bn, D), lambda i: (i, 0)),
            pl.BlockSpec((D, D), lambda i: (0, 0)),
        ],
        out_specs=pl.BlockSpec((bn, D), lambda i: (i, 0)),
        out_shape=jax.ShapeDtypeStruct((NP, D), jnp.float32),
    )(x, w1)


def _tc_scale(xw, deg):
    def body(xw_ref, deg_ref, dinv_ref, y_ref):
        dinv = lax.rsqrt(deg_ref[0] + deg_ref[1] + 1.0)
        dinv_ref[...] = dinv
        y_ref[...] = (dinv * xw_ref[...]).astype(jnp.bfloat16)

    bn = _TC_GRID_BN
    return pl.pallas_call(
        body,
        grid=(NP // bn,),
        in_specs=[
            pl.BlockSpec((bn, D), lambda i: (i, 0)),
            pl.BlockSpec((NC, bn, 1), lambda i: (0, i, 0)),
        ],
        out_specs=[
            pl.BlockSpec((bn, 1), lambda i: (i, 0)),
            pl.BlockSpec((bn, D), lambda i: (i, 0)),
        ],
        out_shape=[
            jax.ShapeDtypeStruct((NP, 1), jnp.float32),
            jax.ShapeDtypeStruct((NP, D), jnp.bfloat16),
        ],
    )(xw, deg)


def _tc_mid(p, y, dinv, b, w, h, hout):
    def body(p_ref, y_ref, dinv_ref, b_ref, w_ref, yout_ref):
        t = (p_ref[0].astype(jnp.float32) + p_ref[1].astype(jnp.float32)
             + y_ref[...].astype(jnp.float32))
        s = dinv_ref[...] * t + b_ref[...]
        hmid = jnp.where(s >= 0, s, 0.2 * s)
        yout_ref[...] = dinv_ref[...] * jnp.dot(hmid, w_ref[...],
                                                preferred_element_type=jnp.float32)

    bn = _TC_GRID_BN
    return pl.pallas_call(
        body,
        grid=(NP // bn,),
        in_specs=[
            pl.BlockSpec((NC, bn, h), lambda i: (0, i, 0)),
            pl.BlockSpec((bn, h), lambda i: (i, 0)),
            pl.BlockSpec((bn, 1), lambda i: (i, 0)),
            pl.BlockSpec((1, h), lambda i: (0, 0)),
            pl.BlockSpec((h, hout), lambda i: (0, 0)),
        ],
        out_specs=pl.BlockSpec((bn, hout), lambda i: (i, 0)),
        out_shape=jax.ShapeDtypeStruct((NP, hout), jnp.float32),
    )(p, y, dinv, b, w)


def _tc_final(p, y, dinv, b, h):
    def body(p_ref, y_ref, dinv_ref, b_ref, out_ref):
        out_ref[...] = (dinv_ref[...] * (p_ref[0] + p_ref[1] + y_ref[...])
                        + b_ref[...])

    bn = _TC_GRID_BN
    return pl.pallas_call(
        body,
        grid=(NP // bn,),
        in_specs=[
            pl.BlockSpec((NC, bn, h), lambda i: (0, i, 0)),
            pl.BlockSpec((bn, h), lambda i: (i, 0)),
            pl.BlockSpec((bn, 1), lambda i: (i, 0)),
            pl.BlockSpec((1, h), lambda i: (0, 0)),
        ],
        out_specs=pl.BlockSpec((bn, h), lambda i: (i, 0)),
        out_shape=jax.ShapeDtypeStruct((NP, h), jnp.float32),
    )(p, y, dinv, b)


def kernel(x, edge_index, W1, b1, W2, b2, W3, b3):
    H1 = W1.shape[1]
    H2 = W2.shape[1]
    C = W3.shape[1]

    e3 = edge_index.reshape(2, NROWS, CH)
    xp = jnp.zeros((NP, D), jnp.float32).at[:N].set(x)
    ones2 = jnp.ones((CH, 1), jnp.float32)
    zdeg = jnp.zeros((NP, 1), jnp.float32)
    zeros128 = jnp.zeros((NP, D), jnp.bfloat16)
    zerosH2 = jnp.zeros((NP, H2), jnp.float32)
    zerosC = jnp.zeros((NP, C), jnp.float32)

    deg = _deg_call(e3, zdeg, ones2)
    xw1 = _tc_mm(xp, W1)
    dinv, y1 = _tc_scale(xw1, deg)

    p = _prop_impl(y1, e3, zeros128, H1, jnp.bfloat16)
    y2 = _tc_mid(p, y1, dinv, b1.reshape(1, H1), W2, H1, H2)

    p = _prop_impl(y2, e3, zerosH2, H2)
    y3 = _tc_mid(p, y2, dinv, b2.reshape(1, H2), W3, H2, C)

    p = _prop_impl(y3, e3, zerosC, C)
    out = _tc_final(p, y3, dinv, b3.reshape(1, C), C)
    return out[:N]
